# Initial kernel scaffold; baseline (speedup 1.0000x reference)
import jax
import jax.numpy as jnp
from jax.experimental import pallas as pl


def kernel(x, W1, b1, W2, b2, W3, b3, W4, b4):
    raise NotImplementedError("write your pallas kernel here")


# speedup vs baseline: 2.0290x; 2.0290x over previous
import jax
import jax.numpy as jnp
from jax import lax
from jax.experimental import pallas as pl
from jax.experimental.pallas import tpu as pltpu

_L = 8192
_NCHUNK = (512 * 512) // _L


def _body(x_ref, w1_ref, w2_ref, w3_ref, w4_ref, b1_ref, b2_ref, b3_ref,
          b4_ref, out_ref, acc_ref):
    w_i = w1_ref[:, 0:1]
    w_j = w1_ref[:, 1:2]
    w_v = w1_ref[:, 2:3]
    b1 = b1_ref[...]
    w2 = w2_ref[...]
    b2 = b2_ref[...]
    w3 = w3_ref[...]
    b3 = b3_ref[...]

    acc_ref[...] = jnp.zeros_like(acc_ref)

    col_iota = lax.broadcasted_iota(jnp.int32, (1, _L), 1)

    def chunk(k, cnt):
        v = x_ref[pl.ds(k, 1), :]
        flat = k * _L + col_iota
        fi = (flat >> 9).astype(jnp.float32)
        fj = (flat & 511).astype(jnp.float32)
        h1 = jnp.maximum(w_i * fi + w_j * fj + w_v * v + b1, 0.0)
        h2 = jnp.maximum(
            jnp.dot(w2, h1, preferred_element_type=jnp.float32) + b2, 0.0)
        h3 = jnp.maximum(
            jnp.dot(w3, h2, preferred_element_type=jnp.float32) + b3, 0.0)
        m = (v != 0.0).astype(jnp.float32)
        acc_ref[...] += h3 * m
        return cnt + jnp.sum(m)

    count = lax.fori_loop(0, _NCHUNK, chunk, jnp.float32(0.0))

    s3 = jnp.sum(acc_ref[...], axis=1, keepdims=True)
    out = jnp.dot(w4_ref[...], s3, preferred_element_type=jnp.float32)
    out_ref[...] = out / count + b4_ref[...]


def kernel(x, W1, b1, W2, b2, W3, b3, W4, b4):
    xv = x.reshape(_NCHUNK, _L)
    out = pl.pallas_call(
        _body,
        out_shape=jax.ShapeDtypeStruct((128, 1), jnp.float32),
        scratch_shapes=[pltpu.VMEM((64, _L), jnp.float32)],
    )(xv, W1, W2, W3, W4,
      b1.reshape(16, 1), b2.reshape(32, 1), b3.reshape(64, 1),
      b4.reshape(128, 1))
    return out.reshape(1, 128)


# speedup vs baseline: 2.2178x; 1.0931x over previous
import jax
import jax.numpy as jnp
from jax import lax
from jax.experimental import pallas as pl

_L = 16384
_NCHUNK = (512 * 512) // _L
_ROWS_PER_CHUNK = _L // 512

_DN = (((1,), (1,)), ((), ()))


def _body(x_ref, w1_ref, w2_ref, w3_ref, w4_ref, b1_ref, b2_ref, b3_ref,
          b4_ref, out_ref):
    w_i = w1_ref[:, 0:1]
    w_j = w1_ref[:, 1:2]
    w_v = w1_ref[:, 2:3]
    w2 = w2_ref[...]
    b2 = b2_ref[...]
    w3 = w3_ref[...]
    b3 = b3_ref[...]

    t = lax.broadcasted_iota(jnp.int32, (1, _L), 1)
    fj = (t & 511).astype(jnp.float32)
    fi0 = (t >> 9).astype(jnp.float32)
    q = w_i * fi0 + w_j * fj + b1_ref[...]

    def chunk(k, carry):
        s3, cnt = carry
        v = x_ref[pl.ds(k, 1), :]
        dq = w_i * (jnp.float32(_ROWS_PER_CHUNK) * k.astype(jnp.float32))
        h1 = jnp.maximum(w_v * v + q + dq, 0.0)
        h2 = jnp.maximum(
            jnp.dot(w2, h1, preferred_element_type=jnp.float32) + b2, 0.0)
        h3 = jnp.maximum(
            jnp.dot(w3, h2, preferred_element_type=jnp.float32) + b3, 0.0)
        m = (v != 0.0).astype(jnp.float32)
        s3 = s3 + lax.dot_general(h3, m, _DN,
                                  preferred_element_type=jnp.float32)
        cnt = cnt + lax.dot_general(m, m, _DN,
                                    preferred_element_type=jnp.float32)
        return s3, cnt

    s3, cnt = lax.fori_loop(
        0, _NCHUNK, chunk,
        (jnp.zeros((64, 1), jnp.float32), jnp.zeros((1, 1), jnp.float32)))

    out = jnp.dot(w4_ref[...], s3, preferred_element_type=jnp.float32)
    out_ref[...] = out / cnt + b4_ref[...]


def kernel(x, W1, b1, W2, b2, W3, b3, W4, b4):
    xv = x.reshape(_NCHUNK, _L)
    out = pl.pallas_call(
        _body,
        out_shape=jax.ShapeDtypeStruct((128, 1), jnp.float32),
    )(xv, W1, W2, W3, W4,
      b1.reshape(16, 1), b2.reshape(32, 1), b3.reshape(64, 1),
      b4.reshape(128, 1))
    return out.reshape(1, 128)


# speedup vs baseline: 2.4851x; 1.1205x over previous
import jax
import jax.numpy as jnp
from jax import lax
from jax.experimental import pallas as pl

_L = 16384
_NCHUNK = (512 * 512) // _L
_ROWS_PER_CHUNK = _L // 512

_DN = (((1,), (1,)), ((), ()))


def _body(x_ref, w1_ref, w2_ref, w3_ref, w4_ref, b1_ref, b2_ref, b3_ref,
          b4_ref, out_ref):
    w_i = w1_ref[:, 0:1]
    w_j = w1_ref[:, 1:2]
    w_v = w1_ref[:, 2:3]
    w2 = w2_ref[...]
    b2 = b2_ref[...]
    w3 = w3_ref[...]
    b3 = b3_ref[...]

    t = lax.broadcasted_iota(jnp.int32, (1, _L), 1)
    fj = (t & 511).astype(jnp.float32)
    fi0 = (t >> 9).astype(jnp.float32)
    q = w_i * fi0 + w_j * fj + b1_ref[...]

    def chunk(k, carry):
        s3, cnt = carry
        v = x_ref[pl.ds(k, 1), :]
        dq = w_i * jnp.float32(_ROWS_PER_CHUNK * k)
        h1 = jnp.maximum(w_v * v + q + dq, 0.0).astype(jnp.bfloat16)
        h2 = jnp.maximum(
            jnp.dot(w2, h1, preferred_element_type=jnp.float32) + b2,
            0.0).astype(jnp.bfloat16)
        h3 = jnp.maximum(
            jnp.dot(w3, h2, preferred_element_type=jnp.float32) + b3, 0.0)
        mf = (v != 0.0).astype(jnp.float32)
        s3 = s3 + lax.dot_general(h3, mf, _DN,
                                  preferred_element_type=jnp.float32)
        cnt = cnt + jnp.sum(mf)
        return s3, cnt

    carry = (jnp.zeros((64, 1), jnp.float32), jnp.float32(0.0))
    for k in range(_NCHUNK):
        carry = chunk(k, carry)
    s3, cnt = carry

    out = jnp.dot(w4_ref[...], s3, preferred_element_type=jnp.float32)
    out_ref[...] = out / cnt + b4_ref[...]


def kernel(x, W1, b1, W2, b2, W3, b3, W4, b4):
    xv = x.reshape(_NCHUNK, _L)
    out = pl.pallas_call(
        _body,
        out_shape=jax.ShapeDtypeStruct((128, 1), jnp.float32),
    )(xv, W1, W2.astype(jnp.bfloat16), W3.astype(jnp.bfloat16), W4,
      b1.reshape(16, 1), b2.reshape(32, 1), b3.reshape(64, 1),
      b4.reshape(128, 1))
    return out.reshape(1, 128)


# speedup vs baseline: 2.9963x; 1.2057x over previous
import jax
import jax.numpy as jnp
from jax import lax
from jax.experimental import pallas as pl

_L = 16384
_NCHUNK = (512 * 512) // _L
_ROWS_PER_CHUNK = _L // 512

_DN = (((1,), (1,)), ((), ()))


def _body(x_ref, w1_ref, w2_ref, w3_ref, w4_ref, b1_ref, b2_ref, b3_ref,
          b4_ref, out_ref):
    w_i = w1_ref[:, 0:1]
    w_j = w1_ref[:, 1:2]
    w_v = w1_ref[:, 2:3]
    w2 = w2_ref[...].astype(jnp.bfloat16)
    b2 = b2_ref[...]
    w3 = w3_ref[...].astype(jnp.bfloat16)
    b3 = b3_ref[...]

    t = lax.broadcasted_iota(jnp.int32, (1, _L), 1)
    fj = (t & 511).astype(jnp.float32)
    fi0 = (t >> 9).astype(jnp.float32)
    q = w_i * fi0 + w_j * fj + b1_ref[...]

    def chunk(k, carry):
        s3, cnt = carry
        v = x_ref[pl.ds(k * _ROWS_PER_CHUNK, _ROWS_PER_CHUNK), :].reshape(
            1, _L)
        dq = w_i * jnp.float32(_ROWS_PER_CHUNK * k)
        h1 = jnp.maximum(w_v * v + q + dq, 0.0).astype(jnp.bfloat16)
        h2 = jnp.maximum(
            jnp.dot(w2, h1, preferred_element_type=jnp.float32) + b2,
            0.0).astype(jnp.bfloat16)
        h3 = jnp.maximum(
            jnp.dot(w3, h2, preferred_element_type=jnp.float32) + b3, 0.0)
        mf = (v != 0.0).astype(jnp.float32)
        s3 = s3 + lax.dot_general(h3, mf, _DN,
                                  preferred_element_type=jnp.float32)
        cnt = cnt + jnp.sum(mf)
        return s3, cnt

    carry = (jnp.zeros((64, 1), jnp.float32), jnp.float32(0.0))
    for k in range(_NCHUNK):
        carry = chunk(k, carry)
    s3, cnt = carry

    out = lax.dot_general(s3 / cnt, w4_ref[...], (((0,), (1,)), ((), ())),
                          preferred_element_type=jnp.float32)
    out_ref[...] = out + b4_ref[...]


def kernel(x, W1, b1, W2, b2, W3, b3, W4, b4):
    return pl.pallas_call(
        _body,
        out_shape=jax.ShapeDtypeStruct((1, 128), jnp.float32),
    )(x, W1, W2, W3, W4,
      b1.reshape(16, 1), b2.reshape(32, 1), b3.reshape(64, 1),
      b4.reshape(1, 128))


# speedup vs baseline: 3.1680x; 1.0573x over previous
import jax
import jax.numpy as jnp
from jax import lax
from jax.experimental import pallas as pl

_L = 16384
_NCHUNK = (512 * 512) // _L
_ROWS_PER_CHUNK = _L // 512

_DN = (((1,), (1,)), ((), ()))


def _body(x_ref, w1_ref, w2_ref, w3_ref, w4_ref, b1_ref, b2_ref, b3_ref,
          b4_ref, out_ref):
    w_i = w1_ref[:, 0:1]
    w_j = w1_ref[:, 1:2]
    w_v = w1_ref[:, 2:3]
    w2 = w2_ref[...].astype(jnp.bfloat16)
    b2 = b2_ref[...].reshape(32, 1)
    w3 = w3_ref[...].astype(jnp.bfloat16)
    b3 = b3_ref[...].reshape(64, 1)

    t = lax.broadcasted_iota(jnp.int32, (1, _L), 1)
    fj = (t & 511).astype(jnp.float32)
    fi0 = (t >> 9).astype(jnp.float32)
    q = w_i * fi0 + w_j * fj + b1_ref[...].reshape(16, 1)

    def chunk(k, carry):
        s3, cnt = carry
        v = x_ref[pl.ds(k * _ROWS_PER_CHUNK, _ROWS_PER_CHUNK), :].reshape(
            1, _L)
        dq = w_i * jnp.float32(_ROWS_PER_CHUNK * k)
        h1 = jnp.maximum(w_v * v + q + dq, 0.0).astype(jnp.bfloat16)
        h2 = jnp.maximum(
            jnp.dot(w2, h1, preferred_element_type=jnp.float32) + b2,
            0.0).astype(jnp.bfloat16)
        h3 = jnp.maximum(
            jnp.dot(w3, h2, preferred_element_type=jnp.float32) + b3, 0.0)
        mf = (v != 0.0).astype(jnp.float32)
        s3 = s3 + lax.dot_general(h3, mf, _DN,
                                  preferred_element_type=jnp.float32)
        cnt = cnt + jnp.sum(mf)
        return s3, cnt

    carry = (jnp.zeros((64, 1), jnp.float32), jnp.float32(0.0))
    for k in range(_NCHUNK):
        carry = chunk(k, carry)
    s3, cnt = carry

    out = lax.dot_general(s3 / cnt, w4_ref[...], (((0,), (1,)), ((), ())),
                          preferred_element_type=jnp.float32)
    out_ref[...] = out + b4_ref[...].reshape(1, 128)


def kernel(x, W1, b1, W2, b2, W3, b3, W4, b4):
    return pl.pallas_call(
        _body,
        out_shape=jax.ShapeDtypeStruct((1, 128), jnp.float32),
    )(x, W1, W2, W3, W4, b1, b2, b3, b4)


# speedup vs baseline: 3.3918x; 1.0706x over previous
import jax
import jax.numpy as jnp
from jax import lax
from jax.experimental import pallas as pl
from jax.experimental.pallas import tpu as pltpu

_L = 16384
_NCHUNK = (512 * 512) // _L
_ROWS_PER_CHUNK = _L // 512

_DN = (((1,), (1,)), ((), ()))


def _body(x_ref, w1_ref, w2_ref, w3_ref, w4_ref, b1_ref, b2_ref, b3_ref,
          b4_ref, out_ref, h1s_ref, h2s_ref):
    w_i = w1_ref[:, 0:1]
    w_j = w1_ref[:, 1:2]
    w_v = w1_ref[:, 2:3]
    w2a = jnp.concatenate(
        [w2_ref[...], b2_ref[...].reshape(32, 1),
         jnp.zeros((32, 7), jnp.float32)], axis=1).astype(jnp.bfloat16)
    w3a = jnp.concatenate(
        [w3_ref[...], b3_ref[...].reshape(64, 1),
         jnp.zeros((64, 7), jnp.float32)], axis=1).astype(jnp.bfloat16)
    h1s_ref[16:24, :] = jnp.zeros((8, _L), jnp.bfloat16)
    h1s_ref[16:17, :] = jnp.ones((1, _L), jnp.bfloat16)
    h2s_ref[32:40, :] = jnp.zeros((8, _L), jnp.bfloat16)
    h2s_ref[32:33, :] = jnp.ones((1, _L), jnp.bfloat16)

    t = lax.broadcasted_iota(jnp.int32, (1, _L), 1)
    fj = (t & 511).astype(jnp.float32)
    fi0 = (t >> 9).astype(jnp.float32)
    q = w_i * fi0 + w_j * fj + b1_ref[...].reshape(16, 1)

    def chunk(k, carry):
        s3, cnt = carry
        v = x_ref[pl.ds(k * _ROWS_PER_CHUNK, _ROWS_PER_CHUNK), :].reshape(
            1, _L)
        dq = w_i * jnp.float32(_ROWS_PER_CHUNK * k)
        h1 = jnp.maximum(w_v * v + q + dq, 0.0).astype(jnp.bfloat16)
        h1s_ref[0:16, :] = h1
        h2 = jnp.maximum(
            jnp.dot(w2a, h1s_ref[...],
                    preferred_element_type=jnp.float32),
            0.0).astype(jnp.bfloat16)
        h2s_ref[0:32, :] = h2
        h3 = jnp.maximum(
            jnp.dot(w3a, h2s_ref[...],
                    preferred_element_type=jnp.float32), 0.0)
        mf = (v != 0.0).astype(jnp.float32)
        s3 = s3 + lax.dot_general(h3, mf, _DN,
                                  preferred_element_type=jnp.float32)
        cnt = cnt + jnp.sum(mf)
        return s3, cnt

    carry = (jnp.zeros((64, 1), jnp.float32), jnp.float32(0.0))
    for k in range(_NCHUNK):
        carry = chunk(k, carry)
    s3, cnt = carry

    out = lax.dot_general(s3 / cnt, w4_ref[...], (((0,), (1,)), ((), ())),
                          preferred_element_type=jnp.float32)
    out_ref[...] = out + b4_ref[...].reshape(1, 128)


def kernel(x, W1, b1, W2, b2, W3, b3, W4, b4):
    return pl.pallas_call(
        _body,
        out_shape=jax.ShapeDtypeStruct((1, 128), jnp.float32),
        scratch_shapes=[pltpu.VMEM((24, _L), jnp.bfloat16),
                        pltpu.VMEM((40, _L), jnp.bfloat16)],
    )(x, W1, W2, W3, W4, b1, b2, b3, b4)
